# Initial kernel scaffold; baseline (speedup 1.0000x reference)
import jax
import jax.numpy as jnp
from jax.experimental import pallas as pl


def kernel(tokens, weight):
    raise NotImplementedError("write your pallas kernel here")


# speedup vs baseline: 1.4218x; 1.4218x over previous
import math

import jax
import jax.numpy as jnp
from jax import lax
from jax.experimental import pallas as pl
from jax.experimental.pallas import tpu as pltpu
from jax.experimental.pallas import tpu_sc as plsc

D = 32
L = 16
NC, NS = 2, 16
NW = NC * NS
S = 128
SCALE = math.sqrt(float(D))


def _emb_kernel(B: int):
    RPW = B // NW
    K = 10
    CH = K * S
    G = RPW // CH

    mesh = plsc.VectorSubcoreMesh(core_axis_name="c", subcore_axis_name="s")

    @pl.kernel(
        out_type=jax.ShapeDtypeStruct((B, D), jnp.float32),
        mesh=mesh,
        compiler_params=pltpu.CompilerParams(use_tc_tiling_on_sc=False),
        scratch_types=[
            pltpu.VMEM((RPW,), jnp.int32),
            pltpu.VMEM((CH, D), jnp.float32),
            pltpu.SemaphoreType.DMA,
        ],
    )
    def body(tok_hbm, w_hbm, out_hbm, idx_v, rows, gsem):
        wid = lax.axis_index("s") * NC + lax.axis_index("c")
        base = wid * RPW
        pltpu.sync_copy(tok_hbm.at[pl.ds(base, RPW)], idx_v)

        @pl.loop(0, G)
        def _chunk(g):
            off = g * CH
            for j in range(K):
                pltpu.async_copy(
                    w_hbm.at[idx_v.at[pl.ds(off + j * S, S)]],
                    rows.at[pl.ds(j * S, S)],
                    gsem,
                )
            for j in range(K):
                pltpu.make_async_copy(
                    w_hbm.at[idx_v.at[pl.ds(off + j * S, S)]],
                    rows.at[pl.ds(j * S, S)],
                    gsem,
                ).wait()

            @pl.loop(0, CH, unroll=8)
            def _scale(i):
                for h in range(D // L):
                    sl = pl.ds(h * L, L)
                    rows[i, sl] = rows[i, sl] * SCALE

            pltpu.sync_copy(rows, out_hbm.at[pl.ds(base + off, CH)])

    return body


def kernel(tokens, weight):
    n0, n1 = tokens.shape
    B = n0 * n1
    flat = tokens.reshape(B).astype(jnp.int32)
    out = _emb_kernel(B)(flat, weight)
    return out.reshape(n0, n1, D)


# speedup vs baseline: 1.4775x; 1.0392x over previous
import math

import jax
import jax.numpy as jnp
from jax import lax
from jax.experimental import pallas as pl
from jax.experimental.pallas import tpu as pltpu
from jax.experimental.pallas import tpu_sc as plsc

D = 32
L = 16
NC, NS = 2, 16
NW = NC * NS
S = 128
SCALE = math.sqrt(float(D))


def _emb_kernel(B: int):
    RPW = B // NW
    K = 10
    CH = K * S
    G = RPW // CH
    assert G >= 4 and G % 2 == 0

    mesh = plsc.VectorSubcoreMesh(core_axis_name="c", subcore_axis_name="s")

    @pl.kernel(
        out_type=jax.ShapeDtypeStruct((B, D), jnp.float32),
        mesh=mesh,
        compiler_params=pltpu.CompilerParams(use_tc_tiling_on_sc=False),
        scratch_types=[
            pltpu.VMEM((RPW,), jnp.int32),
            pltpu.VMEM((CH, D), jnp.float32),
            pltpu.VMEM((CH, D), jnp.float32),
            pltpu.SemaphoreType.DMA,
            pltpu.SemaphoreType.DMA,
            pltpu.SemaphoreType.DMA,
            pltpu.SemaphoreType.DMA,
        ],
    )
    def body(tok_hbm, w_hbm, out_hbm, idx_v, rows0, rows1, g0, g1, o0, o1):
        wid = lax.axis_index("s") * NC + lax.axis_index("c")
        base = wid * RPW
        pltpu.sync_copy(tok_hbm.at[pl.ds(base, RPW)], idx_v)

        rows = (rows0, rows1)
        gsem = (g0, g1)
        osem = (o0, o1)

        def fire(h, p):
            off = h * CH
            for j in range(K):
                pltpu.async_copy(
                    w_hbm.at[idx_v.at[pl.ds(off + j * S, S)]],
                    rows[p].at[pl.ds(j * S, S)],
                    gsem[p],
                )

        def drain(h, p):
            off = h * CH
            for j in range(K):
                pltpu.make_async_copy(
                    w_hbm.at[idx_v.at[pl.ds(off + j * S, S)]],
                    rows[p].at[pl.ds(j * S, S)],
                    gsem[p],
                ).wait()

        def scale(p):
            @pl.loop(0, CH, unroll=8)
            def _scale(i):
                for h in range(D // L):
                    sl = pl.ds(h * L, L)
                    rows[p][i, sl] = rows[p][i, sl] * SCALE

        def out_start(h, p):
            pltpu.async_copy(rows[p], out_hbm.at[pl.ds(base + h * CH, CH)], osem[p])

        def out_wait(h, p):
            pltpu.make_async_copy(
                rows[p], out_hbm.at[pl.ds(base + h * CH, CH)], osem[p]
            ).wait()

        fire(0, 0)
        fire(1, 1)
        drain(0, 0)
        scale(0)
        out_start(0, 0)

        @pl.loop(0, (G - 2) // 2)
        def _pipe(t):
            for b in range(2):
                h = 1 + 2 * t + b
                p = (1 + b) % 2
                q = 1 - p
                out_wait(h - 1, q)
                fire(h + 1, q)
                drain(h, p)
                scale(p)
                out_start(h, p)

        out_wait(G - 2, 0)
        drain(G - 1, 1)
        scale(1)
        out_start(G - 1, 1)
        out_wait(G - 1, 1)

    return body


def kernel(tokens, weight):
    n0, n1 = tokens.shape
    B = n0 * n1
    flat = tokens.reshape(B).astype(jnp.int32)
    out = _emb_kernel(B)(flat, weight)
    return out.reshape(n0, n1, D)


# speedup vs baseline: 1.5876x; 1.0745x over previous
import math

import jax
import jax.numpy as jnp
from jax import lax
from jax.experimental import pallas as pl
from jax.experimental.pallas import tpu as pltpu
from jax.experimental.pallas import tpu_sc as plsc

D = 32
L = 16
NC, NS = 2, 16
NW = NC * NS
S = 128
SCALE = math.sqrt(float(D))


def _emb_kernel(B: int):
    RPW = B // NW
    K = 10
    CH = K * S
    G = RPW // CH
    assert G >= 4 and G % 2 == 0

    mesh = plsc.VectorSubcoreMesh(core_axis_name="c", subcore_axis_name="s")

    @pl.kernel(
        out_type=jax.ShapeDtypeStruct((B, D), jnp.float32),
        mesh=mesh,
        compiler_params=pltpu.CompilerParams(use_tc_tiling_on_sc=False),
        scratch_types=[
            pltpu.VMEM((RPW,), jnp.int32),
            pltpu.VMEM((CH, D), jnp.float32),
            pltpu.VMEM((CH, D), jnp.float32),
            pltpu.SemaphoreType.DMA,
            pltpu.SemaphoreType.DMA,
            pltpu.SemaphoreType.DMA,
            pltpu.SemaphoreType.DMA,
        ],
    )
    def body(tok_hbm, w_hbm, out_hbm, idx_v, rows0, rows1, g0, g1, o0, o1):
        wid = lax.axis_index("s") * NC + lax.axis_index("c")
        base = wid * RPW
        pltpu.sync_copy(tok_hbm.at[pl.ds(base, RPW)], idx_v)

        @pl.loop(0, RPW // L, unroll=8)
        def _remap(i):
            sl = pl.ds(i * L, L)
            v = idx_v[sl]
            idx_v[sl] = ((v >> 11) << 11) + ((v & 511) << 2) + ((v >> 9) & 3)

        rows = (rows0, rows1)
        gsem = (g0, g1)
        osem = (o0, o1)

        def fire(h, p):
            off = h * CH
            for j in range(K):
                pltpu.async_copy(
                    w_hbm.at[idx_v.at[pl.ds(off + j * S, S)]],
                    rows[p].at[pl.ds(j * S, S)],
                    gsem[p],
                )

        def drain(h, p):
            off = h * CH
            for j in range(K):
                pltpu.make_async_copy(
                    w_hbm.at[idx_v.at[pl.ds(off + j * S, S)]],
                    rows[p].at[pl.ds(j * S, S)],
                    gsem[p],
                ).wait()

        def scale(p):
            @pl.loop(0, CH, unroll=8)
            def _scale(i):
                for h in range(D // L):
                    sl = pl.ds(h * L, L)
                    rows[p][i, sl] = rows[p][i, sl] * SCALE

        def out_start(h, p):
            pltpu.async_copy(rows[p], out_hbm.at[pl.ds(base + h * CH, CH)], osem[p])

        def out_wait(h, p):
            pltpu.make_async_copy(
                rows[p], out_hbm.at[pl.ds(base + h * CH, CH)], osem[p]
            ).wait()

        fire(0, 0)
        fire(1, 1)
        drain(0, 0)
        scale(0)
        out_start(0, 0)

        @pl.loop(0, (G - 2) // 2)
        def _pipe(t):
            for b in range(2):
                h = 1 + 2 * t + b
                p = (1 + b) % 2
                q = 1 - p
                out_wait(h - 1, q)
                fire(h + 1, q)
                drain(h, p)
                scale(p)
                out_start(h, p)

        out_wait(G - 2, 0)
        drain(G - 1, 1)
        scale(1)
        out_start(G - 1, 1)
        out_wait(G - 1, 1)

    return body


_TBV = 2048


def _transpose_block(wt_ref, out_ref):
    xt = wt_ref[...].T
    out_ref[...] = jnp.concatenate(
        [xt[512 * k:512 * (k + 1)] for k in range(4)], axis=1
    )


def _repack_weight(weight):
    V = weight.shape[0]
    wt = weight.T
    nblk = (V + _TBV - 1) // _TBV
    w128 = pl.pallas_call(
        _transpose_block,
        grid=(nblk,),
        in_specs=[pl.BlockSpec((D, _TBV), lambda i: (0, i))],
        out_specs=pl.BlockSpec((512, 128), lambda i: (i, 0)),
        out_shape=jax.ShapeDtypeStruct((nblk * 512, 128), jnp.float32),
    )(wt)
    return w128.reshape(nblk * 2048, 32)


def kernel(tokens, weight):
    n0, n1 = tokens.shape
    B = n0 * n1
    flat = tokens.reshape(B).astype(jnp.int32)
    out = _emb_kernel(B)(flat, _repack_weight(weight))
    return out.reshape(n0, n1, D)
